# baseline (speedup 1.0000x reference)
import functools

import jax
import jax.numpy as jnp
from jax import lax
from jax.experimental import pallas as pl
from jax.experimental.pallas import tpu as pltpu
from jax.experimental.pallas import tpu_sc as plsc

N_NODES = 10000
X_DIM = 128
HID = 64
N_GRAPHS = 32

NC = 2
NS = 16
NW = NC * NS
C = 128
EPW = 10000
NCH_E = 80
IDXN = NCH_E * C
ROWS_ACC = N_NODES + 112
RPT_ACC = ROWS_ACC // NS
NBUF = 5

_MESH = plsc.VectorSubcoreMesh(core_axis_name="c", subcore_axis_name="s")
_SC_PARAMS = pltpu.CompilerParams(use_tc_tiling_on_sc=False)


def _fill_tail(idx_v, dummy_base, dummy_n):
    for k in range((IDXN - EPW) // 16):
        lanes = lax.iota(jnp.int32, 16) + (16 * k)
        idx_v[pl.ds(EPW + 16 * k, 16)] = dummy_base + lanes % dummy_n


def _stage_idx(idx_hbm, wid, idx_v, dummy_base, dummy_n):
    base = pl.multiple_of(wid * EPW, 8)
    pltpu.sync_copy(idx_hbm.at[pl.ds(base, EPW)],
                    idx_v.at[pl.ds(0, EPW)])
    _fill_tail(idx_v, dummy_base, dummy_n)


@functools.partial(
    pl.kernel,
    out_type=jax.ShapeDtypeStruct((NC * ROWS_ACC, 16), jnp.float32),
    mesh=_MESH,
    compiler_params=_SC_PARAMS,
    scratch_types=[
        pltpu.VMEM((IDXN,), jnp.int32),
        pltpu.VMEM((C, 16), jnp.float32),
        pltpu.VMEM_SHARED((ROWS_ACC, 16), jnp.float32),
        pltpu.SemaphoreType.DMA,
    ],
)
def _hist_pass(dst_hbm, ones_hbm, zeros_hbm, out_hbm, idx_v, ones_v, acc,
               sem):
    cid = lax.axis_index("c")
    sid = lax.axis_index("s")
    wid = cid * NS + sid
    _stage_idx(dst_hbm, wid, idx_v, N_NODES, 112)
    pltpu.sync_copy(ones_hbm, ones_v)
    pltpu.sync_copy(zeros_hbm.at[pl.ds(sid * RPT_ACC, RPT_ACC)],
                    acc.at[pl.ds(sid * RPT_ACC, RPT_ACC)])
    plsc.subcore_barrier()
    pending = []
    for j in range(NCH_E):
        pending.append(
            pltpu.async_copy(ones_v, acc.at[idx_v.at[pl.ds(j * C, C)]], sem,
                             add=True))
        if len(pending) >= 16:
            for cp in pending:
                cp.wait()
            pending = []
    for cp in pending:
        cp.wait()
    plsc.subcore_barrier()
    pltpu.sync_copy(acc.at[pl.ds(sid * RPT_ACC, RPT_ACC)],
                    out_hbm.at[pl.ds(cid * ROWS_ACC + sid * RPT_ACC, RPT_ACC)])


@functools.partial(
    pl.kernel,
    out_type=jax.ShapeDtypeStruct((NC * ROWS_ACC, HID), jnp.float32),
    mesh=_MESH,
    compiler_params=_SC_PARAMS,
    scratch_types=[
        pltpu.VMEM((IDXN,), jnp.int32),
        pltpu.VMEM((IDXN,), jnp.int32),
        pltpu.VMEM((NBUF, C, HID), jnp.float32),
        pltpu.VMEM_SHARED((ROWS_ACC, HID), jnp.float32),
        pltpu.SemaphoreType.DMA((NBUF,)),
        pltpu.SemaphoreType.DMA((NBUF,)),
    ],
)
def _edge_pass(src_hbm, dst_hbm, y_hbm, zeros_hbm, out_hbm,
               src_v, dst_v, rows, acc, gsem, ssem):
    cid = lax.axis_index("c")
    sid = lax.axis_index("s")
    wid = cid * NS + sid
    base = pl.multiple_of(wid * EPW, 8)
    cs = pltpu.async_copy(src_hbm.at[pl.ds(base, EPW)],
                          src_v.at[pl.ds(0, EPW)], gsem.at[1])
    cd = pltpu.async_copy(dst_hbm.at[pl.ds(base, EPW)],
                          dst_v.at[pl.ds(0, EPW)], gsem.at[2])
    cz = pltpu.async_copy(zeros_hbm.at[pl.ds(sid * RPT_ACC, RPT_ACC)],
                          acc.at[pl.ds(sid * RPT_ACC, RPT_ACC)], gsem.at[3])
    _fill_tail(src_v, 0, N_NODES)
    _fill_tail(dst_v, N_NODES, 112)
    cs.wait()
    cd.wait()
    cz.wait()
    plsc.subcore_barrier()

    def sidx(j):
        return src_v.at[pl.ds(pl.multiple_of(j * C, 8), C)]

    def didx(j):
        return dst_v.at[pl.ds(pl.multiple_of(j * C, 8), C)]

    pltpu.async_copy(y_hbm.at[sidx(0)], rows.at[0], gsem.at[0])

    def step(i, carry):
        for k in range(NBUF):
            j = NBUF * i + k
            b1 = (k + 1) % NBUF

            @pl.when(jnp.logical_and(j >= NBUF - 1, j + 1 < NCH_E))
            def _():
                pltpu.make_async_copy(rows.at[b1], acc.at[didx(0)],
                                      ssem.at[b1]).wait()

            @pl.when(j + 1 < NCH_E)
            def _():
                pltpu.async_copy(y_hbm.at[sidx(j + 1)], rows.at[b1],
                                 gsem.at[b1])

            pltpu.make_async_copy(y_hbm.at[sidx(j)], rows.at[k],
                                  gsem.at[k]).wait()
            pltpu.async_copy(rows.at[k], acc.at[didx(j)], ssem.at[k],
                             add=True)
        return carry

    lax.fori_loop(0, NCH_E // NBUF, step, 0)
    for k in range(NBUF):
        pltpu.make_async_copy(rows.at[k], acc.at[didx(0)],
                              ssem.at[k]).wait()
    plsc.subcore_barrier()
    pltpu.sync_copy(acc.at[pl.ds(sid * RPT_ACC, RPT_ACC)],
                    out_hbm.at[pl.ds(cid * ROWS_ACC + sid * RPT_ACC, RPT_ACC)])


_BR = 2000
_GRID = N_NODES // _BR


def _mm1_body(x_ref, w_ref, o_ref):
    o_ref[...] = jnp.dot(x_ref[...], w_ref[...],
                         preferred_element_type=jnp.float32)


def _mm1(x, w1):
    return pl.pallas_call(
        _mm1_body,
        grid=(_GRID,),
        in_specs=[
            pl.BlockSpec((_BR, X_DIM), lambda i: (i, 0)),
            pl.BlockSpec((X_DIM, HID), lambda i: (0, 0)),
        ],
        out_specs=pl.BlockSpec((_BR, HID), lambda i: (i, 0)),
        out_shape=jax.ShapeDtypeStruct((N_NODES, HID), jnp.float32),
    )(x, w1)


def _post_hist_body(hist_ref, xw_ref, y_ref, dv_ref):
    hp = hist_ref[...]
    dv = lax.rsqrt(hp[0, :, 0:1] + hp[1, :, 0:1] + 1.0)
    dv64 = jnp.broadcast_to(dv, (_BR, HID))
    dv_ref[...] = dv64
    y_ref[...] = xw_ref[...] * dv64


def _post_hist(hist3, xw1):
    return pl.pallas_call(
        _post_hist_body,
        grid=(_GRID,),
        in_specs=[
            pl.BlockSpec((NC, _BR, 16), lambda i: (0, i, 0)),
            pl.BlockSpec((_BR, HID), lambda i: (i, 0)),
        ],
        out_specs=(
            pl.BlockSpec((_BR, HID), lambda i: (i, 0)),
            pl.BlockSpec((_BR, HID), lambda i: (i, 0)),
        ),
        out_shape=(
            jax.ShapeDtypeStruct((N_NODES, HID), jnp.float32),
            jax.ShapeDtypeStruct((N_NODES, HID), jnp.float32),
        ),
    )(hist3, xw1)


def _combine_mm_body(p_ref, y_ref, dv_ref, b_ref, w_ref, yn_ref):
    dv = dv_ref[...]
    h = (p_ref[0] + p_ref[1] + y_ref[...]) * dv + b_ref[...]
    h = jnp.maximum(h, 0.0)
    xwn = jnp.dot(h, w_ref[...], preferred_element_type=jnp.float32)
    yn_ref[...] = xwn * dv


def _combine_mm(parts, y, dv64, b, w_next):
    return pl.pallas_call(
        _combine_mm_body,
        grid=(_GRID,),
        in_specs=[
            pl.BlockSpec((NC, _BR, HID), lambda i: (0, i, 0)),
            pl.BlockSpec((_BR, HID), lambda i: (i, 0)),
            pl.BlockSpec((_BR, HID), lambda i: (i, 0)),
            pl.BlockSpec((1, HID), lambda i: (0, 0)),
            pl.BlockSpec((HID, HID), lambda i: (0, 0)),
        ],
        out_specs=pl.BlockSpec((_BR, HID), lambda i: (i, 0)),
        out_shape=jax.ShapeDtypeStruct((N_NODES, HID), jnp.float32),
    )(parts, y, dv64, b, w_next)


def _pool_final_body(p_ref, y_ref, dv_ref, bat_ref, b_ref, lw_ref, lb_ref,
                     o_ref, acc_g):
    i = pl.program_id(0)
    dv = dv_ref[...]
    h = (p_ref[0] + p_ref[1] + y_ref[...]) * dv + b_ref[...]
    hx = jnp.concatenate([h, jnp.ones((_BR, 1), jnp.float32)], axis=1)
    oh = (bat_ref[...] == lax.broadcasted_iota(jnp.int32, (_BR, N_GRAPHS), 1)
          ).astype(jnp.float32)
    gpart = lax.dot_general(oh, hx, (((0,), (0,)), ((), ())),
                            preferred_element_type=jnp.float32)

    @pl.when(i == 0)
    def _():
        acc_g[...] = jnp.zeros_like(acc_g)

    acc_g[...] += gpart

    a = acc_g[...]
    g = a[:, :HID] * (1.0 / jnp.maximum(a[:, HID:], 1.0))
    z = jnp.dot(g, lw_ref[...], preferred_element_type=jnp.float32)
    o_ref[...] = jax.nn.sigmoid(z + lb_ref[...])


def _pool_final(parts, y, dv64, batch2d, b, lin_w, lin_b):
    return pl.pallas_call(
        _pool_final_body,
        grid=(_GRID,),
        in_specs=[
            pl.BlockSpec((NC, _BR, HID), lambda i: (0, i, 0)),
            pl.BlockSpec((_BR, HID), lambda i: (i, 0)),
            pl.BlockSpec((_BR, HID), lambda i: (i, 0)),
            pl.BlockSpec((_BR, 1), lambda i: (i, 0)),
            pl.BlockSpec((1, HID), lambda i: (0, 0)),
            pl.BlockSpec((HID, 1), lambda i: (0, 0)),
            pl.BlockSpec((1, 1), lambda i: (0, 0)),
        ],
        out_specs=pl.BlockSpec((N_GRAPHS, 1), lambda i: (0, 0)),
        out_shape=jax.ShapeDtypeStruct((N_GRAPHS, 1), jnp.float32),
        scratch_shapes=[
            pltpu.VMEM((N_GRAPHS, HID + 1), jnp.float32),
        ],
    )(parts, y, dv64, batch2d, b, lin_w, lin_b)


def kernel(x, edge_index, batch, W1, b1, W2, b2, W3, b3, lin_W, lin_b):
    edges = edge_index.astype(jnp.int32)
    src = edges[0]
    dst = edges[1]
    batch = batch.astype(jnp.int32)

    ones16 = jnp.ones((C, 16), jnp.float32)
    z_hist = jnp.zeros((ROWS_ACC, 16), jnp.float32)
    z_acc = jnp.zeros((ROWS_ACC, HID), jnp.float32)
    b1r = b1.reshape(1, HID)
    b2r = b2.reshape(1, HID)
    b3r = b3.reshape(1, HID)

    hist = _hist_pass(dst, ones16, z_hist)
    xw1 = _mm1(x, W1)
    hist3 = hist.reshape(NC, ROWS_ACC, 16)
    y1, dv64 = _post_hist(hist3, xw1)

    p1 = _edge_pass(src, dst, y1, z_acc)
    y2 = _combine_mm(p1.reshape(NC, ROWS_ACC, HID), y1, dv64,
                     b1r, W2)
    p2 = _edge_pass(src, dst, y2, z_acc)
    y3 = _combine_mm(p2.reshape(NC, ROWS_ACC, HID), y2, dv64,
                     b2r, W3)
    p3 = _edge_pass(src, dst, y3, z_acc)

    out = _pool_final(p3.reshape(NC, ROWS_ACC, HID), y3, dv64,
                      batch.reshape(N_NODES, 1), b3r,
                      lin_W, lin_b.reshape(1, 1))
    return out
